# baseline (speedup 1.0000x reference)
import jax
import jax.numpy as jnp
from jax.experimental import pallas as pl
from jax.experimental.pallas import tpu as pltpu

ALPHA = 0.2
NEG_BIG = -9000000000000000.0
BLK = 256

_NT_DIMS = (((1,), (1,)), ((), ()))


def _evalue_block(na, hwt_bf, arow, arow_swap):
    mp = (na > 0).astype(jnp.bfloat16)
    mm = (na < 0).astype(jnp.bfloat16)
    hpt = jax.lax.dot_general(hwt_bf, mp, _NT_DIMS,
                              preferred_element_type=jnp.float32)
    hmt = jax.lax.dot_general(hwt_bf, mm, _NT_DIMS,
                              preferred_element_type=jnp.float32)
    ait = jnp.concatenate([hpt, hmt], axis=0).astype(jnp.bfloat16)
    e_p = jnp.dot(arow, ait, preferred_element_type=jnp.float32)
    e_m = jnp.dot(arow_swap, ait, preferred_element_type=jnp.float32)
    ep = jnp.where(e_p >= 0, e_p, ALPHA * e_p)
    em = jnp.where(e_m >= 0, e_m, ALPHA * e_m)
    return ep, em


def _attn_block(ea, ep, em, hw_bf):
    gt = ea > 0
    lt = ea < 0
    la = jnp.where(gt, ep, jnp.where(lt, em, NEG_BIG))
    m = jnp.max(la, axis=1, keepdims=True)
    lb = jnp.where(lt, em, jnp.where(gt, ep, NEG_BIG))
    p = jnp.exp(lb - m)
    inv = 1.0 / jnp.sum(p, axis=1, keepdims=True)
    w = ((p * inv) * ea).astype(jnp.bfloat16)
    return jnp.dot(w, hw_bf, preferred_element_type=jnp.float32)


def _make_fused_kernel(nb, blk, n, out_f):
    def _fused(h_ref, w_ref, arow_ref, arow_swap_ref,
               na0_ref, na1_ref, ea0_ref, ea1_ref,
               out0_ref, out1_ref,
               hw_bf_scr, hwt_bf_scr, ep_scr, em_scr):
        i = pl.program_id(0)

        @pl.when(i == 0)
        def _proj():
            hw = jnp.dot(h_ref[...].astype(jnp.bfloat16),
                         w_ref[...].astype(jnp.bfloat16),
                         preferred_element_type=jnp.float32)
            hw_bf = hw.astype(jnp.bfloat16)
            hw_bf_scr[...] = hw_bf
            hwt_bf_scr[...] = jnp.transpose(hw_bf)

        @pl.when((i >= 1) & (i <= nb))
        def _ev():
            hwt_bf = hwt_bf_scr[...]
            arow = arow_ref[...]
            arow_swap = arow_swap_ref[...]
            col = (i - 1) * blk
            ep0, em0 = _evalue_block(na0_ref[...], hwt_bf, arow, arow_swap)
            ep_scr[0:1, pl.ds(col, blk)] = ep0
            em_scr[0:1, pl.ds(col, blk)] = em0
            ep1, em1 = _evalue_block(na1_ref[...], hwt_bf, arow, arow_swap)
            ep_scr[0:1, pl.ds(col + n // 2, blk)] = ep1
            em_scr[0:1, pl.ds(col + n // 2, blk)] = em1

        @pl.when(i >= nb + 1)
        def _at():
            ep = ep_scr[...]
            em = em_scr[...]
            hw_bf = hw_bf_scr[...]
            out0_ref[...] = _attn_block(ea0_ref[...], ep, em, hw_bf)
            out1_ref[...] = _attn_block(ea1_ref[...], ep, em, hw_bf)

    return _fused


def kernel(h, node_adj, edge_adj, W, a):
    n, in_f = h.shape
    out_f = W.shape[1]
    blk = BLK
    nb = n // (2 * blk)

    a_bf = a.astype(jnp.bfloat16)
    arow = a_bf.reshape(1, 2 * out_f)
    arow_swap = jnp.concatenate(
        [a_bf[out_f:], a_bf[:out_f]], axis=0).reshape(1, 2 * out_f)

    na_idx = lambda i: (jnp.clip(i - 1, 0, nb - 1), 0)
    na1_idx = lambda i: (jnp.clip(i - 1, 0, nb - 1) + nb, 0)
    ea_idx = lambda i: (jnp.clip(i - 1 - nb, 0, nb - 1), 0)
    ea1_idx = lambda i: (jnp.clip(i - 1 - nb, 0, nb - 1) + nb, 0)

    out_lo, out_hi = pl.pallas_call(
        _make_fused_kernel(nb, blk, n, out_f),
        grid=(2 * nb + 1,),
        in_specs=[
            pl.BlockSpec((n, in_f), lambda i: (0, 0)),
            pl.BlockSpec((in_f, out_f), lambda i: (0, 0)),
            pl.BlockSpec((1, 2 * out_f), lambda i: (0, 0)),
            pl.BlockSpec((1, 2 * out_f), lambda i: (0, 0)),
            pl.BlockSpec((blk, n), na_idx),
            pl.BlockSpec((blk, n), na1_idx),
            pl.BlockSpec((blk, n), ea_idx),
            pl.BlockSpec((blk, n), ea1_idx),
        ],
        out_specs=[
            pl.BlockSpec((blk, out_f), ea_idx),
            pl.BlockSpec((blk, out_f), ea_idx),
        ],
        out_shape=[
            jax.ShapeDtypeStruct((n // 2, out_f), jnp.float32),
            jax.ShapeDtypeStruct((n // 2, out_f), jnp.float32),
        ],
        scratch_shapes=[
            pltpu.VMEM((n, out_f), jnp.bfloat16),
            pltpu.VMEM((out_f, n), jnp.bfloat16),
            pltpu.VMEM((1, n), jnp.float32),
            pltpu.VMEM((1, n), jnp.float32),
        ],
        compiler_params=pltpu.CompilerParams(
            dimension_semantics=("arbitrary",),
            vmem_limit_bytes=64 * 1024 * 1024),
    )(h, W, arow, arow_swap, node_adj, node_adj, edge_adj, edge_adj)

    return jnp.concatenate([out_lo, out_hi], axis=0)


# speedup vs baseline: 1.1895x; 1.1895x over previous
import jax
import jax.numpy as jnp
from jax.experimental import pallas as pl
from jax.experimental.pallas import tpu as pltpu

ALPHA = 0.2
NEG_BIG = -9000000000000000.0
BLK = 256

_NT_DIMS = (((1,), (1,)), ((), ()))


def _evalue_block(na, hwt_bf, arow, arow_swap):
    mp = (na > 0).astype(jnp.bfloat16)
    mm = (na < 0).astype(jnp.bfloat16)
    hpt = jax.lax.dot_general(hwt_bf, mp, _NT_DIMS,
                              preferred_element_type=jnp.float32)
    hmt = jax.lax.dot_general(hwt_bf, mm, _NT_DIMS,
                              preferred_element_type=jnp.float32)
    ait = jnp.concatenate([hpt, hmt], axis=0).astype(jnp.bfloat16)
    e_p = jnp.dot(arow, ait, preferred_element_type=jnp.float32)
    e_m = jnp.dot(arow_swap, ait, preferred_element_type=jnp.float32)
    ep = jnp.where(e_p >= 0, e_p, ALPHA * e_p)
    em = jnp.where(e_m >= 0, e_m, ALPHA * e_m)
    return ep, em


def _attn_block(ea, ep, em, hw_bf):
    gt = ea > 0
    lt = ea < 0
    logits = jnp.where(gt, ep, jnp.where(lt, em, NEG_BIG))
    m = jnp.max(logits, axis=1, keepdims=True)
    p = jnp.exp(logits - m)
    inv = 1.0 / jnp.sum(p, axis=1, keepdims=True)
    w = ((p * inv) * ea).astype(jnp.bfloat16)
    return jnp.dot(w, hw_bf, preferred_element_type=jnp.float32)


def _make_fused_kernel(nb, blk, n, out_f):
    def _fused(h_ref, w_ref, arow_ref, arow_swap_ref,
               na0_ref, na1_ref, ea0_ref, ea1_ref,
               out0_ref, out1_ref,
               hw_bf_scr, hwt_bf_scr, ep_scr, em_scr):
        i = pl.program_id(0)

        @pl.when(i == 0)
        def _proj():
            hw = jnp.dot(h_ref[...].astype(jnp.bfloat16),
                         w_ref[...].astype(jnp.bfloat16),
                         preferred_element_type=jnp.float32)
            hw_bf = hw.astype(jnp.bfloat16)
            hw_bf_scr[...] = hw_bf
            hwt_bf_scr[...] = jnp.transpose(hw_bf)

        @pl.when((i >= 1) & (i <= nb))
        def _ev():
            hwt_bf = hwt_bf_scr[...]
            arow = arow_ref[...]
            arow_swap = arow_swap_ref[...]
            col = (i - 1) * blk
            ep0, em0 = _evalue_block(na0_ref[...], hwt_bf, arow, arow_swap)
            ep_scr[0:1, pl.ds(col, blk)] = ep0
            em_scr[0:1, pl.ds(col, blk)] = em0
            ep1, em1 = _evalue_block(na1_ref[...], hwt_bf, arow, arow_swap)
            ep_scr[0:1, pl.ds(col + n // 2, blk)] = ep1
            em_scr[0:1, pl.ds(col + n // 2, blk)] = em1

        @pl.when(i >= nb + 1)
        def _at():
            ep = ep_scr[...]
            em = em_scr[...]
            hw_bf = hw_bf_scr[...]
            out0_ref[...] = _attn_block(ea0_ref[...], ep, em, hw_bf)
            out1_ref[...] = _attn_block(ea1_ref[...], ep, em, hw_bf)

    return _fused


def kernel(h, node_adj, edge_adj, W, a):
    n, in_f = h.shape
    out_f = W.shape[1]
    blk = BLK
    nb = n // (2 * blk)

    a_bf = a.astype(jnp.bfloat16)
    arow = a_bf.reshape(1, 2 * out_f)
    arow_swap = jnp.concatenate(
        [a_bf[out_f:], a_bf[:out_f]], axis=0).reshape(1, 2 * out_f)

    na_idx = lambda i: (jnp.clip(i - 1, 0, nb - 1), 0)
    na1_idx = lambda i: (jnp.clip(i - 1, 0, nb - 1) + nb, 0)
    ea_idx = lambda i: (jnp.clip(i - 1 - nb, 0, nb - 1), 0)
    ea1_idx = lambda i: (jnp.clip(i - 1 - nb, 0, nb - 1) + nb, 0)

    out_lo, out_hi = pl.pallas_call(
        _make_fused_kernel(nb, blk, n, out_f),
        grid=(2 * nb + 1,),
        in_specs=[
            pl.BlockSpec((n, in_f), lambda i: (0, 0)),
            pl.BlockSpec((in_f, out_f), lambda i: (0, 0)),
            pl.BlockSpec((1, 2 * out_f), lambda i: (0, 0)),
            pl.BlockSpec((1, 2 * out_f), lambda i: (0, 0)),
            pl.BlockSpec((blk, n), na_idx),
            pl.BlockSpec((blk, n), na1_idx),
            pl.BlockSpec((blk, n), ea_idx),
            pl.BlockSpec((blk, n), ea1_idx),
        ],
        out_specs=[
            pl.BlockSpec((blk, out_f), ea_idx),
            pl.BlockSpec((blk, out_f), ea_idx),
        ],
        out_shape=[
            jax.ShapeDtypeStruct((n // 2, out_f), jnp.float32),
            jax.ShapeDtypeStruct((n // 2, out_f), jnp.float32),
        ],
        scratch_shapes=[
            pltpu.VMEM((n, out_f), jnp.bfloat16),
            pltpu.VMEM((out_f, n), jnp.bfloat16),
            pltpu.VMEM((1, n), jnp.float32),
            pltpu.VMEM((1, n), jnp.float32),
        ],
        compiler_params=pltpu.CompilerParams(
            dimension_semantics=("arbitrary",),
            vmem_limit_bytes=64 * 1024 * 1024),
    )(h, W, arow, arow_swap, node_adj, node_adj, edge_adj, edge_adj)

    return jnp.concatenate([out_lo, out_hi], axis=0)


# speedup vs baseline: 1.2072x; 1.0149x over previous
import jax
import jax.numpy as jnp
from jax.experimental import pallas as pl
from jax.experimental.pallas import tpu as pltpu

ALPHA = 0.2
NEG_BIG = -9000000000000000.0
BLK = 256

_NT_DIMS = (((1,), (1,)), ((), ()))


def _evalue_block(na, hwt_bf, arow, arow_swap):
    mp = (na > 0).astype(jnp.bfloat16)
    mm = (na < 0).astype(jnp.bfloat16)
    hpt = jax.lax.dot_general(hwt_bf, mp, _NT_DIMS,
                              preferred_element_type=jnp.float32)
    hmt = jax.lax.dot_general(hwt_bf, mm, _NT_DIMS,
                              preferred_element_type=jnp.float32)
    ait = jnp.concatenate([hpt, hmt], axis=0).astype(jnp.bfloat16)
    e_p = jnp.dot(arow, ait, preferred_element_type=jnp.float32)
    e_m = jnp.dot(arow_swap, ait, preferred_element_type=jnp.float32)
    ep = jnp.where(e_p >= 0, e_p, ALPHA * e_p)
    em = jnp.where(e_m >= 0, e_m, ALPHA * e_m)
    return ep, em


def _attn_block(ea, ep, em, hw_bf):
    gt = ea > 0
    lt = ea < 0
    logits = jnp.where(gt, ep, jnp.where(lt, em, NEG_BIG))
    m = jnp.max(logits, axis=1, keepdims=True)
    p = jnp.exp(logits - m)
    inv = 1.0 / jnp.sum(p, axis=1, keepdims=True)
    w = ((p * inv) * ea).astype(jnp.bfloat16)
    return jnp.dot(w, hw_bf, preferred_element_type=jnp.float32)


def _make_fused_kernel(nb, blk, n, out_f):
    def _fused(h_ref, w_ref, arow_ref, arow_swap_ref,
               na0_ref, na1_ref, ea0_ref, ea1_ref,
               out_ref,
               hw_bf_scr, hwt_bf_scr, ep_scr, em_scr):
        i = pl.program_id(0)

        @pl.when(i == 0)
        def _proj():
            hw = jnp.dot(h_ref[...].astype(jnp.bfloat16),
                         w_ref[...].astype(jnp.bfloat16),
                         preferred_element_type=jnp.float32)
            hw_bf = hw.astype(jnp.bfloat16)
            hw_bf_scr[...] = hw_bf
            hwt_bf_scr[...] = jnp.transpose(hw_bf)

        @pl.when((i >= 1) & (i <= nb))
        def _ev():
            hwt_bf = hwt_bf_scr[...]
            arow = arow_ref[...]
            arow_swap = arow_swap_ref[...]
            col = (i - 1) * 2 * blk
            ep0, em0 = _evalue_block(na0_ref[...], hwt_bf, arow, arow_swap)
            ep_scr[0:1, pl.ds(col, blk)] = ep0
            em_scr[0:1, pl.ds(col, blk)] = em0
            ep1, em1 = _evalue_block(na1_ref[...], hwt_bf, arow, arow_swap)
            ep_scr[0:1, pl.ds(col + blk, blk)] = ep1
            em_scr[0:1, pl.ds(col + blk, blk)] = em1

        @pl.when(i >= nb + 1)
        def _at():
            ep = ep_scr[...]
            em = em_scr[...]
            hw_bf = hw_bf_scr[...]
            out_ref[0:blk, :] = _attn_block(ea0_ref[...], ep, em, hw_bf)
            out_ref[blk:2 * blk, :] = _attn_block(ea1_ref[...], ep, em, hw_bf)

    return _fused


def kernel(h, node_adj, edge_adj, W, a):
    n, in_f = h.shape
    out_f = W.shape[1]
    blk = BLK
    nb = n // (2 * blk)

    a_bf = a.astype(jnp.bfloat16)
    arow = a_bf.reshape(1, 2 * out_f)
    arow_swap = jnp.concatenate(
        [a_bf[out_f:], a_bf[:out_f]], axis=0).reshape(1, 2 * out_f)

    na_idx = lambda i: (2 * jnp.clip(i - 1, 0, nb - 1), 0)
    na1_idx = lambda i: (2 * jnp.clip(i - 1, 0, nb - 1) + 1, 0)
    ea_idx = lambda i: (2 * jnp.clip(i - 1 - nb, 0, nb - 1), 0)
    ea1_idx = lambda i: (2 * jnp.clip(i - 1 - nb, 0, nb - 1) + 1, 0)
    out_idx = lambda i: (jnp.clip(i - 1 - nb, 0, nb - 1), 0)

    h_prime = pl.pallas_call(
        _make_fused_kernel(nb, blk, n, out_f),
        grid=(2 * nb + 1,),
        in_specs=[
            pl.BlockSpec((n, in_f), lambda i: (0, 0)),
            pl.BlockSpec((in_f, out_f), lambda i: (0, 0)),
            pl.BlockSpec((1, 2 * out_f), lambda i: (0, 0)),
            pl.BlockSpec((1, 2 * out_f), lambda i: (0, 0)),
            pl.BlockSpec((blk, n), na_idx),
            pl.BlockSpec((blk, n), na1_idx),
            pl.BlockSpec((blk, n), ea_idx),
            pl.BlockSpec((blk, n), ea1_idx),
        ],
        out_specs=pl.BlockSpec((2 * blk, out_f), out_idx),
        out_shape=jax.ShapeDtypeStruct((n, out_f), jnp.float32),
        scratch_shapes=[
            pltpu.VMEM((n, out_f), jnp.bfloat16),
            pltpu.VMEM((out_f, n), jnp.bfloat16),
            pltpu.VMEM((1, n), jnp.float32),
            pltpu.VMEM((1, n), jnp.float32),
        ],
        compiler_params=pltpu.CompilerParams(
            dimension_semantics=("arbitrary",),
            vmem_limit_bytes=64 * 1024 * 1024),
    )(h, W, arow, arow_swap, node_adj, node_adj, edge_adj, edge_adj)

    return h_prime
